# baseline (speedup 1.0000x reference)
import jax
import jax.numpy as jnp
from jax import lax
from jax.experimental import pallas as pl
from jax.experimental.pallas import tpu as pltpu

VOCAB = 1000000
EMBED_DIM = 64
BATCH = 16384
NUM_CLASSES = 128

BM = 2048
NB = BATCH // BM
HALF = BM // 2
UNROLL = 4


def _body(idx_sref, emb_ref, w_ref, b_ref, lab_ref, out_ref, stage_ref,
          xbuf, sems_d, sems_s, sem_x):
    i = pl.program_id(0)

    def issue_block(block, slot):
        def issue_one(j, _):
            r0 = idx_sref[block * BM + j]
            pltpu.make_async_copy(
                emb_ref.at[pl.ds(r0, 1), :],
                xbuf.at[slot, pl.ds(j, 1), :],
                sems_d.at[slot],
            ).start()
            r1 = idx_sref[block * BM + HALF + j]
            pltpu.make_async_copy(
                emb_ref.at[pl.ds(r1, 1), :],
                stage_ref.at[slot, pl.ds(j, 1), :],
                sems_s.at[slot],
            ).start()
            return 0
        lax.fori_loop(0, HALF, issue_one, 0, unroll=UNROLL)

    @pl.when(i == 0)
    def _():
        issue_block(0, 0)

    @pl.when(i + 1 < NB)
    def _():
        issue_block(i + 1, (i + 1) % 2)

    slot = i % 2
    pltpu.make_async_copy(
        emb_ref.at[pl.ds(0, HALF), :], stage_ref.at[slot], sems_s.at[slot]
    ).wait()
    big = pltpu.make_async_copy(
        stage_ref.at[slot], xbuf.at[slot, pl.ds(HALF, HALF), :], sem_x
    )
    big.start()
    pltpu.make_async_copy(
        emb_ref.at[pl.ds(0, HALF), :],
        xbuf.at[slot, pl.ds(0, HALF), :],
        sems_d.at[slot],
    ).wait()
    big.wait()

    x = xbuf[slot]
    w = w_ref[...]
    bias = b_ref[...]
    lab = lab_ref[0, 0, :]
    scores = jnp.dot(x, w, preferred_element_type=jnp.float32) + bias
    m = jnp.max(scores, axis=-1, keepdims=True)
    lse = jnp.log(jnp.sum(jnp.exp(scores - m), axis=-1, keepdims=True)) + m
    cls = lax.broadcasted_iota(jnp.int32, scores.shape, 1)
    picked = jnp.sum(
        jnp.where(cls == lab[:, None], scores, 0.0), axis=-1, keepdims=True
    )
    part = jnp.sum(lse - picked, axis=0, keepdims=True) * (1.0 / BATCH)

    @pl.when(i == 0)
    def _():
        out_ref[...] = part

    @pl.when(i > 0)
    def _():
        out_ref[...] = out_ref[...] + part


def kernel(sentence_features, labels, emb, W, b):
    idx = sentence_features.astype(jnp.int32)
    labels3 = labels.astype(jnp.int32).reshape(NB, 1, BM)
    grid_spec = pltpu.PrefetchScalarGridSpec(
        num_scalar_prefetch=1,
        grid=(NB,),
        in_specs=[
            pl.BlockSpec(memory_space=pltpu.HBM),
            pl.BlockSpec((EMBED_DIM, NUM_CLASSES), lambda i, *_: (0, 0)),
            pl.BlockSpec((1, NUM_CLASSES), lambda i, *_: (0, 0)),
            pl.BlockSpec((1, 1, BM), lambda i, *_: (i, 0, 0)),
        ],
        out_specs=[
            pl.BlockSpec((1, 1), lambda i, *_: (0, 0)),
            pl.BlockSpec(memory_space=pltpu.HBM),
        ],
        scratch_shapes=[
            pltpu.VMEM((2, BM, EMBED_DIM), jnp.float32),
            pltpu.SemaphoreType.DMA((2,)),
            pltpu.SemaphoreType.DMA((2,)),
            pltpu.SemaphoreType.DMA,
        ],
    )
    loss, _ = pl.pallas_call(
        _body,
        grid_spec=grid_spec,
        out_shape=[
            jax.ShapeDtypeStruct((1, 1), jnp.float32),
            jax.ShapeDtypeStruct((2, HALF, EMBED_DIM), jnp.float32),
        ],
    )(idx, emb, W, b.reshape(1, NUM_CLASSES), labels3)
    return loss[0, 0]


# speedup vs baseline: 1.1571x; 1.1571x over previous
import jax
import jax.numpy as jnp
from jax import lax
from jax.experimental import pallas as pl
from jax.experimental.pallas import tpu as pltpu

VOCAB = 1000000
EMBED_DIM = 64
BATCH = 16384
NUM_CLASSES = 128

BM = 2048
NB = BATCH // BM
HALF = BM // 2
UNROLL = 4


def _body(idx_sref, emb_ref, w_ref, b_ref, lab_ref, out_ref,
          xbuf, ybuf, sems_x, sems_y):
    i = pl.program_id(0)

    def issue_block(block, slot):
        def issue_one(j, _):
            r0 = idx_sref[block * BM + j]
            pltpu.make_async_copy(
                emb_ref.at[pl.ds(r0, 1), :],
                xbuf.at[slot, pl.ds(j, 1), :],
                sems_x.at[slot],
            ).start()
            r1 = idx_sref[block * BM + HALF + j]
            pltpu.make_async_copy(
                emb_ref.at[pl.ds(r1, 1), :],
                ybuf.at[slot, pl.ds(j, 1), :],
                sems_y.at[slot],
            ).start()
            return 0
        lax.fori_loop(0, HALF, issue_one, 0, unroll=UNROLL)

    @pl.when(i == 0)
    def _():
        issue_block(0, 0)

    @pl.when(i + 1 < NB)
    def _():
        issue_block(i + 1, (i + 1) % 2)

    slot = i % 2
    pltpu.make_async_copy(
        emb_ref.at[pl.ds(0, HALF), :], xbuf.at[slot], sems_x.at[slot]
    ).wait()
    pltpu.make_async_copy(
        emb_ref.at[pl.ds(0, HALF), :], ybuf.at[slot], sems_y.at[slot]
    ).wait()

    w = w_ref[...]
    bias = b_ref[...]
    part = jnp.zeros((1, 1), jnp.float32)
    for half, buf in ((0, xbuf), (1, ybuf)):
        x = buf[slot]
        lab = lab_ref[0, 0, pl.ds(half * HALF, HALF)]
        scores = jnp.dot(x, w, preferred_element_type=jnp.float32) + bias
        m = jnp.max(scores, axis=-1, keepdims=True)
        lse = jnp.log(jnp.sum(jnp.exp(scores - m), axis=-1, keepdims=True)) + m
        cls = lax.broadcasted_iota(jnp.int32, scores.shape, 1)
        picked = jnp.sum(
            jnp.where(cls == lab[:, None], scores, 0.0), axis=-1, keepdims=True
        )
        part = part + jnp.sum(lse - picked, axis=0, keepdims=True)
    part = part * (1.0 / BATCH)

    @pl.when(i == 0)
    def _():
        out_ref[...] = part

    @pl.when(i > 0)
    def _():
        out_ref[...] = out_ref[...] + part


def kernel(sentence_features, labels, emb, W, b):
    idx = sentence_features.astype(jnp.int32)
    labels3 = labels.astype(jnp.int32).reshape(NB, 1, BM)
    grid_spec = pltpu.PrefetchScalarGridSpec(
        num_scalar_prefetch=1,
        grid=(NB,),
        in_specs=[
            pl.BlockSpec(memory_space=pltpu.HBM),
            pl.BlockSpec((EMBED_DIM, NUM_CLASSES), lambda i, *_: (0, 0)),
            pl.BlockSpec((1, NUM_CLASSES), lambda i, *_: (0, 0)),
            pl.BlockSpec((1, 1, BM), lambda i, *_: (i, 0, 0)),
        ],
        out_specs=pl.BlockSpec((1, 1), lambda i, *_: (0, 0)),
        scratch_shapes=[
            pltpu.VMEM((2, HALF, EMBED_DIM), jnp.float32),
            pltpu.VMEM((2, HALF, EMBED_DIM), jnp.float32),
            pltpu.SemaphoreType.DMA((2,)),
            pltpu.SemaphoreType.DMA((2,)),
        ],
    )
    loss = pl.pallas_call(
        _body,
        grid_spec=grid_spec,
        out_shape=jax.ShapeDtypeStruct((1, 1), jnp.float32),
    )(idx, emb, W, b.reshape(1, NUM_CLASSES), labels3)
    return loss[0, 0]


# speedup vs baseline: 1.1812x; 1.0209x over previous
import jax
import jax.numpy as jnp
from jax import lax
from jax.experimental import pallas as pl
from jax.experimental.pallas import tpu as pltpu

VOCAB = 1000000
EMBED_DIM = 64
BATCH = 16384
NUM_CLASSES = 128

BM = 2048
NB = BATCH // BM
HALF = BM // 2
UNROLL = 4


def _body(idx_sref, emb_ref, w_ref, b_ref, lab_ref, out_ref,
          xbuf, ybuf, sems_x, sems_y):
    i = pl.program_id(0)

    def issue_block(block, slot):
        def issue_one(j, _):
            r0 = idx_sref[block * BM + j]
            pltpu.make_async_copy(
                emb_ref.at[pl.ds(r0, 1), :],
                xbuf.at[slot, pl.ds(j, 1), :],
                sems_x.at[slot],
            ).start(priority=0)
            r1 = idx_sref[block * BM + HALF + j]
            pltpu.make_async_copy(
                emb_ref.at[pl.ds(r1, 1), :],
                ybuf.at[slot, pl.ds(j, 1), :],
                sems_y.at[slot],
            ).start(priority=1)
            return 0
        lax.fori_loop(0, HALF, issue_one, 0, unroll=UNROLL)

    @pl.when(i == 0)
    def _():
        issue_block(0, 0)

    @pl.when(i + 1 < NB)
    def _():
        issue_block(i + 1, (i + 1) % 2)

    slot = i % 2
    pltpu.make_async_copy(
        emb_ref.at[pl.ds(0, HALF), :], xbuf.at[slot], sems_x.at[slot]
    ).wait()
    pltpu.make_async_copy(
        emb_ref.at[pl.ds(0, HALF), :], ybuf.at[slot], sems_y.at[slot]
    ).wait()

    w = w_ref[...]
    bias = b_ref[...]
    part = jnp.zeros((1, 1), jnp.float32)
    for half, buf in ((0, xbuf), (1, ybuf)):
        x = buf[slot]
        lab = lab_ref[0, 0, pl.ds(half * HALF, HALF)]
        scores = jnp.dot(x, w, preferred_element_type=jnp.float32) + bias
        m = jnp.max(scores, axis=-1, keepdims=True)
        lse = jnp.log(jnp.sum(jnp.exp(scores - m), axis=-1, keepdims=True)) + m
        cls = lax.broadcasted_iota(jnp.int32, scores.shape, 1)
        picked = jnp.sum(
            jnp.where(cls == lab[:, None], scores, 0.0), axis=-1, keepdims=True
        )
        part = part + jnp.sum(lse - picked, axis=0, keepdims=True)
    part = part * (1.0 / BATCH)

    @pl.when(i == 0)
    def _():
        out_ref[...] = part

    @pl.when(i > 0)
    def _():
        out_ref[...] = out_ref[...] + part


def kernel(sentence_features, labels, emb, W, b):
    idx = sentence_features.astype(jnp.int32)
    labels3 = labels.astype(jnp.int32).reshape(NB, 1, BM)
    grid_spec = pltpu.PrefetchScalarGridSpec(
        num_scalar_prefetch=1,
        grid=(NB,),
        in_specs=[
            pl.BlockSpec(memory_space=pltpu.HBM),
            pl.BlockSpec((EMBED_DIM, NUM_CLASSES), lambda i, *_: (0, 0)),
            pl.BlockSpec((1, NUM_CLASSES), lambda i, *_: (0, 0)),
            pl.BlockSpec((1, 1, BM), lambda i, *_: (i, 0, 0)),
        ],
        out_specs=pl.BlockSpec((1, 1), lambda i, *_: (0, 0)),
        scratch_shapes=[
            pltpu.VMEM((2, HALF, EMBED_DIM), jnp.float32),
            pltpu.VMEM((2, HALF, EMBED_DIM), jnp.float32),
            pltpu.SemaphoreType.DMA((2,)),
            pltpu.SemaphoreType.DMA((2,)),
        ],
    )
    loss = pl.pallas_call(
        _body,
        grid_spec=grid_spec,
        out_shape=jax.ShapeDtypeStruct((1, 1), jnp.float32),
    )(idx, emb, W, b.reshape(1, NUM_CLASSES), labels3)
    return loss[0, 0]
